# baseline (speedup 1.0000x reference)
import functools

import jax
import jax.numpy as jnp
from jax import lax
from jax.experimental import pallas as pl
from jax.experimental.pallas import tpu as pltpu
from jax.experimental.pallas import tpu_sc as plsc

_N = 500000
_M = 32
_KSUB = 256
_DSUB = 4
_D = _M * _DSUB
_NC = 2
_NS = 16
_NW = _NC * _NS
_RPW = _N // _NW
_R = 125
_NCHUNK = _RPW // _R
_SEG = 136


def _decode_body(codes_hbm, tbl_hbm, out_hbm, tbl_v,
                 ca, cb, oa, ob, sia, sib, soa, sob):
    wid = lax.axis_index("s") * _NC + lax.axis_index("c")
    pltpu.sync_copy(tbl_hbm, tbl_v)

    iota = lax.iota(jnp.int32, 16)
    colbase = [iota * _SEG + h * 16 * _SEG for h in range(2)]

    base_row = wid * _RPW

    def seg_start(k):
        row0 = base_row + k * _R
        s = jnp.minimum((row0 >> 3) << 3, _N - _SEG)
        return row0, pl.multiple_of(s, 8)

    def start_in(k, cv, sem):
        _, s = seg_start(k)
        for m in range(_M):
            pltpu.async_copy(
                codes_hbm.at[pl.ds(m * _N + s, _SEG)],
                cv.at[pl.ds(m * _SEG, _SEG)], sem)

    def wait_in(cv, sem):
        for m in range(_M):
            pltpu.make_async_copy(
                codes_hbm.at[pl.ds(m * _N, _SEG)],
                cv.at[pl.ds(m * _SEG, _SEG)], sem).wait()

    def start_out(k, ov, sem):
        row0 = base_row + k * _R
        pltpu.async_copy(ov, out_hbm.at[pl.ds(row0 * _D, _R * _D)], sem)

    def wait_out(ov, sem):
        pltpu.make_async_copy(
            ov, out_hbm.at[pl.ds(0, _R * _D)], sem).wait()

    def compute(k, cv, ov):
        row0, s = seg_start(k)
        off = row0 - s

        @plsc.parallel_loop(0, _R, unroll=2)
        def row_body(r):
            rv = jnp.full((16,), off + r, jnp.int32)
            for h in range(2):
                ch = plsc.load_gather(cv, [colbase[h] + rv])
                base4 = (ch << 2) + (iota << 10) + (h * 16384)
                sbase = jnp.full((16,), r * _D + h * 64,
                                 jnp.int32) + (iota << 2)
                for d in range(4):
                    val = plsc.load_gather(tbl_v, [base4 + d])
                    plsc.store_scatter(ov, [sbase + d], val)

    start_in(0, ca, sia)
    wait_in(ca, sia)
    start_in(1, cb, sib)
    compute(0, ca, oa)
    start_out(0, oa, soa)
    wait_in(cb, sib)
    start_in(2, ca, sia)
    compute(1, cb, ob)
    start_out(1, ob, sob)

    def pair_body(g, _):
        k0 = 2 * g
        wait_in(ca, sia)
        start_in(k0 + 1, cb, sib)
        wait_out(oa, soa)
        compute(k0, ca, oa)
        start_out(k0, oa, soa)
        wait_in(cb, sib)
        start_in(k0 + 2, ca, sia)
        wait_out(ob, sob)
        compute(k0 + 1, cb, ob)
        start_out(k0 + 1, ob, sob)
        return 0

    lax.fori_loop(1, _NCHUNK // 2, pair_body, 0)

    wait_in(ca, sia)
    wait_out(oa, soa)
    compute(_NCHUNK - 1, ca, oa)
    start_out(_NCHUNK - 1, oa, soa)
    wait_out(oa, soa)
    wait_out(ob, sob)


_mesh = plsc.VectorSubcoreMesh(core_axis_name="c", subcore_axis_name="s")

_decode = functools.partial(
    pl.kernel,
    mesh=_mesh,
    compiler_params=pltpu.CompilerParams(
        use_tc_tiling_on_sc=False, needs_layout_passes=False),
    out_type=jax.ShapeDtypeStruct((_N * _D,), jnp.float32),
    scratch_types=[
        pltpu.VMEM((_M * _KSUB * _DSUB,), jnp.float32),
        pltpu.VMEM((_M * _SEG,), jnp.int32),
        pltpu.VMEM((_M * _SEG,), jnp.int32),
        pltpu.VMEM((_R * _D,), jnp.float32),
        pltpu.VMEM((_R * _D,), jnp.float32),
        pltpu.SemaphoreType.DMA,
        pltpu.SemaphoreType.DMA,
        pltpu.SemaphoreType.DMA,
        pltpu.SemaphoreType.DMA,
    ],
)(_decode_body)


@jax.jit
def kernel(codes, centroids):
    out = _decode(codes.T.reshape(-1), centroids.reshape(-1))
    return out.reshape(_N, _D)


# speedup vs baseline: 1.3054x; 1.3054x over previous
import functools

import jax
import jax.numpy as jnp
from jax import lax
from jax.experimental import pallas as pl
from jax.experimental.pallas import tpu as pltpu
from jax.experimental.pallas import tpu_sc as plsc

_N = 500000
_M = 32
_KSUB = 256
_DSUB = 4
_D = _M * _DSUB
_W = _M // 4
_NC = 2
_NS = 16
_NW = _NC * _NS
_RPW = _N // _NW
_R = 125
_NCHUNK = _RPW // _R


def _decode_body(codes_hbm, tbl_hbm, out_hbm, tbl_v,
                 ca, cb, oa, ob, sia, sib, soa, sob):
    wid = lax.axis_index("s") * _NC + lax.axis_index("c")
    pltpu.sync_copy(tbl_hbm, tbl_v)

    iota = lax.iota(jnp.int32, 16)
    wordsel = iota >> 2
    bytesh = (iota & 3) << 3
    base_row = wid * _RPW

    def start_in(k, cv, sem):
        row0 = base_row + k * _R
        pltpu.async_copy(
            codes_hbm.at[pl.ds(row0 * _W, _R * _W)], cv, sem)

    def wait_in(cv, sem):
        pltpu.make_async_copy(
            codes_hbm.at[pl.ds(0, _R * _W)], cv, sem).wait()

    def start_out(k, ov, sem):
        row0 = base_row + k * _R
        pltpu.async_copy(ov, out_hbm.at[pl.ds(row0 * _D, _R * _D)], sem)

    def wait_out(ov, sem):
        pltpu.make_async_copy(
            ov, out_hbm.at[pl.ds(0, _R * _D)], sem).wait()

    def compute(cv, ov):
        @plsc.parallel_loop(0, _R, unroll=2)
        def row_body(r):
            for h in range(2):
                w = plsc.load_gather(
                    cv, [jnp.full((16,), r * _W + 4 * h, jnp.int32)
                         + wordsel])
                ch = (w >> bytesh) & 255
                base4 = (ch << 2) + (iota << 10) + (h * 16384)
                sbase = jnp.full((16,), r * _D + h * 64,
                                 jnp.int32) + (iota << 2)
                for d in range(4):
                    val = plsc.load_gather(tbl_v, [base4 + d])
                    plsc.store_scatter(ov, [sbase + d], val)

    start_in(0, ca, sia)
    wait_in(ca, sia)
    start_in(1, cb, sib)
    compute(ca, oa)
    start_out(0, oa, soa)
    wait_in(cb, sib)
    start_in(2, ca, sia)
    compute(cb, ob)
    start_out(1, ob, sob)

    def pair_body(g, _):
        k0 = 2 * g
        wait_in(ca, sia)
        start_in(k0 + 1, cb, sib)
        wait_out(oa, soa)
        compute(ca, oa)
        start_out(k0, oa, soa)
        wait_in(cb, sib)
        start_in(k0 + 2, ca, sia)
        wait_out(ob, sob)
        compute(cb, ob)
        start_out(k0 + 1, ob, sob)
        return 0

    lax.fori_loop(1, _NCHUNK // 2, pair_body, 0)

    wait_in(ca, sia)
    wait_out(oa, soa)
    compute(ca, oa)
    start_out(_NCHUNK - 1, oa, soa)
    wait_out(oa, soa)
    wait_out(ob, sob)


_mesh = plsc.VectorSubcoreMesh(core_axis_name="c", subcore_axis_name="s")

_decode = functools.partial(
    pl.kernel,
    mesh=_mesh,
    compiler_params=pltpu.CompilerParams(
        use_tc_tiling_on_sc=False, needs_layout_passes=False),
    out_type=jax.ShapeDtypeStruct((_N * _D,), jnp.float32),
    scratch_types=[
        pltpu.VMEM((_M * _KSUB * _DSUB,), jnp.float32),
        pltpu.VMEM((_R * _W,), jnp.int32),
        pltpu.VMEM((_R * _W,), jnp.int32),
        pltpu.VMEM((_R * _D,), jnp.float32),
        pltpu.VMEM((_R * _D,), jnp.float32),
        pltpu.SemaphoreType.DMA,
        pltpu.SemaphoreType.DMA,
        pltpu.SemaphoreType.DMA,
        pltpu.SemaphoreType.DMA,
    ],
)(_decode_body)


@jax.jit
def kernel(codes, centroids):
    packed = lax.bitcast_convert_type(
        codes.astype(jnp.int8).reshape(_N, _W, 4), jnp.int32)
    out = _decode(packed.reshape(-1), centroids.reshape(-1))
    return out.reshape(_N, _D)


# speedup vs baseline: 2.2120x; 1.6944x over previous
import functools

import jax
import jax.numpy as jnp
from jax import lax
from jax.experimental import pallas as pl
from jax.experimental.pallas import tpu as pltpu
from jax.experimental.pallas import tpu_sc as plsc

_N = 500000
_M = 32
_KSUB = 256
_DSUB = 4
_D = _M * _DSUB
_NC = 2
_NS = 16
_NW = _NC * _NS
_RPW = _N // _NW
_R = 125
_NCHUNK = _RPW // _R


def _decode_body(codes_hbm, tbl_hbm, out_hbm, tbl_v,
                 ca, cb, oa, ob, sia, sib, soa, sob):
    wid = lax.axis_index("s") * _NC + lax.axis_index("c")
    pltpu.sync_copy(tbl_hbm, tbl_v)

    iota = lax.iota(jnp.int32, 16)
    base_row = wid * _RPW

    def start_in(k, cv, sem):
        row0 = base_row + k * _R
        pltpu.async_copy(codes_hbm.at[pl.ds(row0, _R)], cv, sem)

    def wait_in(cv, sem):
        pltpu.make_async_copy(
            codes_hbm.at[pl.ds(0, _R)], cv, sem).wait()

    def start_out(k, ov, sem):
        row0 = base_row + k * _R
        pltpu.async_copy(ov, out_hbm.at[pl.ds(row0, _R)], sem)

    def wait_out(ov, sem):
        pltpu.make_async_copy(
            ov, out_hbm.at[pl.ds(0, _R)], sem).wait()

    def compute(cv, ov):
        @plsc.parallel_loop(0, _R, unroll=2)
        def row_body(r):
            rv = jnp.full((16,), r, jnp.int32)
            for h in range(2):
                ch = plsc.load_gather(cv, [rv, iota + h * 16])
                base4 = (ch << 2) + (iota << 10) + (h * 16384)
                sbase = (iota << 2) + h * 64
                for d in range(4):
                    val = plsc.load_gather(tbl_v, [base4 + d])
                    plsc.store_scatter(ov, [rv, sbase + d], val)

    start_in(0, ca, sia)
    wait_in(ca, sia)
    start_in(1, cb, sib)
    compute(ca, oa)
    start_out(0, oa, soa)
    wait_in(cb, sib)
    start_in(2, ca, sia)
    compute(cb, ob)
    start_out(1, ob, sob)

    def pair_body(g, _):
        k0 = 2 * g
        wait_in(ca, sia)
        start_in(k0 + 1, cb, sib)
        wait_out(oa, soa)
        compute(ca, oa)
        start_out(k0, oa, soa)
        wait_in(cb, sib)
        start_in(k0 + 2, ca, sia)
        wait_out(ob, sob)
        compute(cb, ob)
        start_out(k0 + 1, ob, sob)
        return 0

    lax.fori_loop(1, _NCHUNK // 2, pair_body, 0)

    wait_in(ca, sia)
    wait_out(oa, soa)
    compute(ca, oa)
    start_out(_NCHUNK - 1, oa, soa)
    wait_out(oa, soa)
    wait_out(ob, sob)


_mesh = plsc.VectorSubcoreMesh(core_axis_name="c", subcore_axis_name="s")

_decode = functools.partial(
    pl.kernel,
    mesh=_mesh,
    compiler_params=pltpu.CompilerParams(
        use_tc_tiling_on_sc=False, needs_layout_passes=False),
    out_type=jax.ShapeDtypeStruct((_N, _D), jnp.float32),
    scratch_types=[
        pltpu.VMEM((_M * _KSUB * _DSUB,), jnp.float32),
        pltpu.VMEM((_R, _M), jnp.int32),
        pltpu.VMEM((_R, _M), jnp.int32),
        pltpu.VMEM((_R, _D), jnp.float32),
        pltpu.VMEM((_R, _D), jnp.float32),
        pltpu.SemaphoreType.DMA,
        pltpu.SemaphoreType.DMA,
        pltpu.SemaphoreType.DMA,
        pltpu.SemaphoreType.DMA,
    ],
)(_decode_body)


@jax.jit
def kernel(codes, centroids):
    return _decode(codes, centroids.reshape(-1))


# speedup vs baseline: 2.7520x; 1.2441x over previous
import functools

import jax
import jax.numpy as jnp
from jax import lax
from jax.experimental import pallas as pl
from jax.experimental.pallas import tpu as pltpu
from jax.experimental.pallas import tpu_sc as plsc

_N = 500000
_M = 32
_KSUB = 256
_DSUB = 4
_D = _M * _DSUB
_NC = 2
_NS = 16
_NW = _NC * _NS
_RPW = 15624
_R = 168
_NCHUNK = _RPW // _R
_TAIL = _N - _RPW * _NW


def _decode_body(codes_hbm, tbl_hbm, out_hbm, tbl_v,
                 ca, cb, oa, ob, sia, sib, soa, sob):
    wid = lax.axis_index("s") * _NC + lax.axis_index("c")
    pltpu.sync_copy(tbl_hbm, tbl_v)

    iota = lax.iota(jnp.int32, 16)
    base_row = wid * _RPW

    def start_in(k, cv, sem):
        row0 = pl.multiple_of(base_row + k * _R, 8)
        pltpu.async_copy(codes_hbm.at[pl.ds(row0, _R)], cv, sem)

    def wait_in(cv, sem):
        pltpu.make_async_copy(
            codes_hbm.at[pl.ds(0, _R)], cv, sem).wait()

    def start_out(k, ov, sem):
        row0 = pl.multiple_of(base_row + k * _R, 8)
        pltpu.async_copy(ov, out_hbm.at[pl.ds(row0, _R)], sem)

    def wait_out(ov, sem):
        pltpu.make_async_copy(
            ov, out_hbm.at[pl.ds(0, _R)], sem).wait()

    def compute(cv, ov, nrows):
        @plsc.parallel_loop(0, nrows, unroll=2)
        def row_body(r):
            rv = jnp.full((16,), r, jnp.int32)
            for h in range(2):
                ch = plsc.load_gather(cv, [rv, iota + h * 16])
                base4 = (ch << 2) + (iota << 10) + (h * 16384)
                sbase = (iota << 2) + h * 64
                for d in range(4):
                    val = plsc.load_gather(tbl_v, [base4 + d])
                    plsc.store_scatter(ov, [rv, sbase + d], val)

    start_in(0, ca, sia)
    wait_in(ca, sia)
    start_in(1, cb, sib)
    compute(ca, oa, _R)
    start_out(0, oa, soa)
    wait_in(cb, sib)
    start_in(2, ca, sia)
    compute(cb, ob, _R)
    start_out(1, ob, sob)

    def pair_body(g, _):
        k0 = 2 * g
        wait_in(ca, sia)
        start_in(k0 + 1, cb, sib)
        wait_out(oa, soa)
        compute(ca, oa, _R)
        start_out(k0, oa, soa)
        wait_in(cb, sib)
        start_in(k0 + 2, ca, sia)
        wait_out(ob, sob)
        compute(cb, ob, _R)
        start_out(k0 + 1, ob, sob)
        return 0

    lax.fori_loop(1, _NCHUNK // 2, pair_body, 0)

    wait_in(ca, sia)
    wait_out(oa, soa)
    compute(ca, oa, _R)
    start_out(_NCHUNK - 1, oa, soa)
    wait_out(oa, soa)
    wait_out(ob, sob)

    @pl.when(wid == _NW - 1)
    def _tail():
        t0 = _RPW * _NW
        pltpu.sync_copy(codes_hbm.at[pl.ds(t0, _TAIL)],
                        ca.at[pl.ds(0, _TAIL)])
        compute(ca, oa, _TAIL)
        pltpu.sync_copy(oa.at[pl.ds(0, _TAIL)],
                        out_hbm.at[pl.ds(t0, _TAIL)])


_mesh = plsc.VectorSubcoreMesh(core_axis_name="c", subcore_axis_name="s")

_decode = functools.partial(
    pl.kernel,
    mesh=_mesh,
    compiler_params=pltpu.CompilerParams(
        use_tc_tiling_on_sc=True, needs_layout_passes=False),
    out_type=jax.ShapeDtypeStruct((_N, _D), jnp.float32),
    scratch_types=[
        pltpu.VMEM((_M * _KSUB * _DSUB,), jnp.float32),
        pltpu.VMEM((_R, _M), jnp.int32),
        pltpu.VMEM((_R, _M), jnp.int32),
        pltpu.VMEM((_R, _D), jnp.float32),
        pltpu.VMEM((_R, _D), jnp.float32),
        pltpu.SemaphoreType.DMA,
        pltpu.SemaphoreType.DMA,
        pltpu.SemaphoreType.DMA,
        pltpu.SemaphoreType.DMA,
    ],
)(_decode_body)


@jax.jit
def kernel(codes, centroids):
    return _decode(codes, centroids.reshape(-1))
